# baseline (speedup 1.0000x reference)
import functools

import jax
import jax.numpy as jnp
from jax import lax
from jax.experimental import pallas as pl
from jax.experimental.pallas import tpu as pltpu
from jax.experimental.pallas import tpu_sc as plsc


def _make_gather(V, D, B):
    info = plsc.get_sparse_core_info()
    NC, NS = info.num_cores, info.num_subcores
    NW = NC * NS
    assert B % (8 * NW) == 0 and V % 8 == 0
    b_per_w = B // NW
    C = 32
    n_chunks = b_per_w // C
    mesh = plsc.VectorSubcoreMesh(core_axis_name="c", subcore_axis_name="s")

    @functools.partial(
        pl.kernel,
        mesh=mesh,
        out_type=jax.ShapeDtypeStruct((B // 2, 2 * D), jnp.float32),
        scratch_types=[
            pltpu.VMEM((b_per_w,), jnp.int32),
            pltpu.VMEM((C, 8, D), jnp.float32),
            pltpu.VMEM((C, 8, D), jnp.float32),
            pltpu.VMEM((C // 2, 2 * D), jnp.float32),
            pltpu.SemaphoreType.DMA,
            pltpu.SemaphoreType.DMA,
        ],
    )
    def gather_kernel(y_hbm, table_hbm, out_hbm, y_v, g0, g1, rows_v,
                      sem0, sem1):
        wid = lax.axis_index("s") * NC + lax.axis_index("c")
        base = wid * b_per_w
        pltpu.sync_copy(y_hbm.at[pl.ds(base, b_per_w)], y_v)

        table3 = table_hbm.reshape(V // 8, 8, D)

        def fire(c, g, sm):
            @pl.loop(0, C // 16)
            def _(k):
                t = y_v[pl.ds(c * C + k * 16, 16)] >> 3
                for j in range(16):
                    pltpu.async_copy(table3.at[t[j]], g.at[k * 16 + j], sm)

        def drain_extract_store(c, g, sm):
            pltpu.make_async_copy(table3.at[pl.ds(0, C)], g, sm).wait()

            @pl.loop(0, C // 16)
            def _(k):
                s = y_v[pl.ds(c * C + k * 16, 16)] & 7
                for j in range(16):
                    sj = s[j]
                    for u in range(D // 16):
                        rows_v[k * 8 + j // 2,
                               pl.ds((j % 2) * D + u * 16, 16)] = (
                            g[k * 16 + j, sj, pl.ds(u * 16, 16)]
                        )

            off = pl.multiple_of(base // 2 + c * (C // 2), 8)
            pltpu.sync_copy(rows_v, out_hbm.at[pl.ds(off, C // 2)])

        fire(0, g0, sem0)

        @pl.loop(0, n_chunks // 2)
        def _(m):
            c0 = 2 * m
            fire(c0 + 1, g1, sem1)
            drain_extract_store(c0, g0, sem0)

            @pl.when(c0 + 2 < n_chunks)
            def _():
                fire(c0 + 2, g0, sem0)

            drain_extract_store(c0 + 1, g1, sem1)

    return gather_kernel


@jax.jit
def kernel(y, table):
    B, = y.shape
    V, D = table.shape
    out2 = _make_gather(V, D, B)(y.astype(jnp.int32), table)
    return out2.reshape(B, D)
